# baseline (speedup 1.0000x reference)
import jax
import jax.numpy as jnp
from jax.experimental import pallas as pl

VOCAB = 8192
HALF = VOCAB // 2
EMBED = 32
BLKM = 256


def _bf16_half_down(x):
    i = jax.lax.bitcast_convert_type(x, jnp.int32)
    i = (i + jnp.int32(0x7FFF)) & jnp.int32(-65536)
    return jax.lax.bitcast_convert_type(i, jnp.float32)


def _half_argmin(dist, base):
    m = jnp.min(dist, axis=1, keepdims=True)
    col = jax.lax.broadcasted_iota(jnp.int32, (BLKM, HALF), 1)
    idx = jnp.min(jnp.where(dist == m, col, HALF), axis=1) + base
    return m[:, 0], idx.astype(jnp.int32)


def _vq_block(z_ref, cb_ref, tok_ref, zq_ref):
    zb = z_ref[...]
    cb = cb_ref[...]
    zb16 = zb.astype(jnp.bfloat16)
    cb16 = cb.astype(jnp.bfloat16)
    sq = zb * zb
    z2 = sq[:, 0:1]
    for k in range(1, EMBED):
        z2 = z2 + sq[:, k:k + 1]
    c2 = jnp.sum(cb * cb, axis=1)[None, :]
    dots = jax.lax.dot_general(zb16, cb16, (((1,), (1,)), ((), ())),
                               preferred_element_type=jnp.float32)
    dist = (z2 + c2) - 2.0 * dots
    m0, i0 = _half_argmin(dist[:, :HALF], 0)
    m1, i1 = _half_argmin(dist[:, HALF:], HALF)
    tok = jnp.where(m1 < _bf16_half_down(m0), i1, i0)
    tok_ref[0, 0, :] = tok
    col = jax.lax.broadcasted_iota(jnp.int32, (BLKM, VOCAB), 1)
    onehot = (col == tok[:, None]).astype(jnp.float32)
    zq_ref[...] = jax.lax.dot_general(
        onehot, cb, (((1,), (0,)), ((), ())),
        preferred_element_type=jnp.float32,
        precision=jax.lax.Precision.HIGHEST)


def kernel(z, codebook):
    b, e, h, w = z.shape
    m = b * h * w
    nblk = m // BLKM
    z_flat = jnp.transpose(z, (0, 2, 3, 1)).reshape(m, e)
    tok3, zq_flat = pl.pallas_call(
        _vq_block,
        grid=(nblk,),
        in_specs=[
            pl.BlockSpec((BLKM, e), lambda i: (i, 0)),
            pl.BlockSpec((VOCAB, e), lambda i: (0, 0)),
        ],
        out_specs=[
            pl.BlockSpec((1, 1, BLKM), lambda i: (i, 0, 0)),
            pl.BlockSpec((BLKM, e), lambda i: (i, 0)),
        ],
        out_shape=[
            jax.ShapeDtypeStruct((nblk, 1, BLKM), jnp.int32),
            jax.ShapeDtypeStruct((m, e), jnp.float32),
        ],
    )(z_flat, codebook)
    tokens = tok3.reshape(b, h * w)
    z_q = jnp.transpose(zq_flat.reshape(b, h, w, e), (0, 3, 1, 2))
    decoder_input = z + jax.lax.stop_gradient(z_q - z)
    return (z, z_q, tokens, decoder_input)


# speedup vs baseline: 2.0998x; 2.0998x over previous
import jax
import jax.numpy as jnp
from jax.experimental import pallas as pl

VOCAB = 8192
HALF = VOCAB // 2
EMBED = 32
BLKM = 256


def _bf16_half_down(x):
    i = jax.lax.bitcast_convert_type(x, jnp.int32)
    i = (i + jnp.int32(0x7FFF)) & jnp.int32(-65536)
    return jax.lax.bitcast_convert_type(i, jnp.float32)


def _half_argmin(dist, base):
    m = jnp.min(dist, axis=1, keepdims=True)
    col = jax.lax.broadcasted_iota(jnp.int32, (BLKM, HALF), 1)
    idx = jnp.min(jnp.where(dist == m, col, HALF), axis=1) + base
    return m[:, 0], idx.astype(jnp.int32)


def _vq_block(z_ref, cb_ref, tok_ref, zq_ref):
    zb = z_ref[...]
    cb = cb_ref[...]
    zb16 = zb.astype(jnp.bfloat16)
    cb16 = cb.astype(jnp.bfloat16)
    sq = zb * zb
    z2 = sq[:, 0:1]
    for k in range(1, EMBED):
        z2 = z2 + sq[:, k:k + 1]
    c2 = jnp.sum(cb * cb, axis=1)[None, :]
    dots = jax.lax.dot_general(zb16, cb16, (((1,), (1,)), ((), ())),
                               preferred_element_type=jnp.float32)
    dist = (z2 + c2) - 2.0 * dots
    m0, i0 = _half_argmin(dist[:, :HALF], 0)
    m1, i1 = _half_argmin(dist[:, HALF:], HALF)
    tok = jnp.where(m1 < _bf16_half_down(m0), i1, i0)
    tok_ref[0, 0, :] = tok
    col = jax.lax.broadcasted_iota(jnp.int32, (BLKM, VOCAB), 1)
    onehot = (col == tok[:, None]).astype(jnp.bfloat16)
    zq_ref[...] = jax.lax.dot_general(
        onehot, cb16, (((1,), (0,)), ((), ())),
        preferred_element_type=jnp.float32)


def kernel(z, codebook):
    b, e, h, w = z.shape
    m = b * h * w
    nblk = m // BLKM
    z_flat = jnp.transpose(z, (0, 2, 3, 1)).reshape(m, e)
    tok3, zq_flat = pl.pallas_call(
        _vq_block,
        grid=(nblk,),
        in_specs=[
            pl.BlockSpec((BLKM, e), lambda i: (i, 0)),
            pl.BlockSpec((VOCAB, e), lambda i: (0, 0)),
        ],
        out_specs=[
            pl.BlockSpec((1, 1, BLKM), lambda i: (i, 0, 0)),
            pl.BlockSpec((BLKM, e), lambda i: (i, 0)),
        ],
        out_shape=[
            jax.ShapeDtypeStruct((nblk, 1, BLKM), jnp.int32),
            jax.ShapeDtypeStruct((m, e), jnp.float32),
        ],
    )(z_flat, codebook)
    tokens = tok3.reshape(b, h * w)
    z_q = jnp.transpose(zq_flat.reshape(b, h, w, e), (0, 3, 1, 2))
    decoder_input = z + jax.lax.stop_gradient(z_q - z)
    return (z, z_q, tokens, decoder_input)


# speedup vs baseline: 2.3214x; 1.1055x over previous
import functools

import jax
import jax.numpy as jnp
from jax import lax
from jax.experimental import pallas as pl
from jax.experimental.pallas import tpu as pltpu
from jax.experimental.pallas import tpu_sc as plsc

VOCAB = 8192
HALF = VOCAB // 2
EMBED = 32
BLKM = 256

_SC_INFO = plsc.get_sparse_core_info()
_NW = _SC_INFO.num_cores * _SC_INFO.num_subcores
_B_PER_W = 16384 // _NW


def _bf16_half_down(x):
    i = lax.bitcast_convert_type(x, jnp.int32)
    i = (i + jnp.int32(0x7FFF)) & jnp.int32(-65536)
    return lax.bitcast_convert_type(i, jnp.float32)


def _half_argmin(dist, base):
    m = jnp.min(dist, axis=1, keepdims=True)
    col = lax.broadcasted_iota(jnp.int32, (BLKM, HALF), 1)
    idx = jnp.min(jnp.where(dist == m, col, HALF), axis=1) + base
    return m[:, 0], idx.astype(jnp.int32)


def _vq_block(z_ref, cb_ref, tok_ref):
    zb = z_ref[...]
    cb = cb_ref[...]
    zb16 = zb.astype(jnp.bfloat16)
    cb16 = cb.astype(jnp.bfloat16)
    sq = zb * zb
    z2 = sq[:, 0:1]
    for k in range(1, EMBED):
        z2 = z2 + sq[:, k:k + 1]
    c2 = jnp.sum(cb * cb, axis=1)[None, :]
    dots = lax.dot_general(zb16, cb16, (((1,), (1,)), ((), ())),
                           preferred_element_type=jnp.float32)
    dist = (z2 + c2) - 2.0 * dots
    m0, i0 = _half_argmin(dist[:, :HALF], 0)
    m1, i1 = _half_argmin(dist[:, HALF:], HALF)
    tok_ref[0, 0, :] = jnp.where(m1 < _bf16_half_down(m0), i1, i0)


@functools.partial(
    pl.kernel,
    mesh=plsc.VectorSubcoreMesh(core_axis_name="c", subcore_axis_name="s"),
    out_type=jax.ShapeDtypeStruct((16384, 128), jnp.float32),
    scratch_types=[
        pltpu.VMEM((_B_PER_W,), jnp.int32),
        pltpu.VMEM((_B_PER_W, 128), jnp.float32),
        pltpu.SemaphoreType.DMA,
    ],
)
def _sc_gather(table_hbm, idx_hbm, out_hbm, idx_v, rows_v, sem):
    wid = lax.axis_index("s") * _SC_INFO.num_cores + lax.axis_index("c")
    base = wid * _B_PER_W
    pltpu.sync_copy(idx_hbm.at[pl.ds(base, _B_PER_W)], idx_v)
    pltpu.async_copy(table_hbm.at[idx_v], rows_v, sem).wait()
    pltpu.sync_copy(rows_v, out_hbm.at[pl.ds(base, _B_PER_W)])


def kernel(z, codebook):
    b, e, h, w = z.shape
    m = b * h * w
    nblk = m // BLKM
    z_flat = jnp.transpose(z, (0, 2, 3, 1)).reshape(m, e)
    tok3 = pl.pallas_call(
        _vq_block,
        grid=(nblk,),
        in_specs=[
            pl.BlockSpec((BLKM, e), lambda i: (i, 0)),
            pl.BlockSpec((VOCAB, e), lambda i: (0, 0)),
        ],
        out_specs=pl.BlockSpec((1, 1, BLKM), lambda i: (i, 0, 0)),
        out_shape=jax.ShapeDtypeStruct((nblk, 1, BLKM), jnp.int32),
    )(z_flat, codebook)
    tokens_flat = tok3.reshape(m)
    cb_pad = jnp.pad(codebook, ((0, 0), (0, 128 - e)))
    zq_flat = _sc_gather(cb_pad, tokens_flat)[:, :e]
    tokens = tok3.reshape(b, h * w)
    z_q = jnp.transpose(zq_flat.reshape(b, h, w, e), (0, 3, 1, 2))
    decoder_input = z + lax.stop_gradient(z_q - z)
    return (z, z_q, tokens, decoder_input)
